# baseline (speedup 1.0000x reference)
import functools
import math

import jax
import jax.numpy as jnp
from jax import lax
from jax.experimental import pallas as pl
from jax.experimental.pallas import tpu as pltpu
from jax.experimental.pallas import tpu_sc as plsc

_NUM_BUCKETS = 32
_MAX_DISTANCE = 128
_N_HEADS = 16
_QLEN = 2048
_KLEN = 2048
_NSHIFT = 32
_TEXT = 4224
_TWIDTH = 4064


def _table_body(delta_ref, w_ref, out_ref):
    m = lax.broadcasted_iota(jnp.int32, (1, _TEXT), 1)
    rel = m - (_QLEN - 1) + delta_ref[0, 0]
    n = -rel
    half = _NUM_BUCKETS // 2
    ret = jnp.where(n < 0, half, 0).astype(jnp.int32)
    n = jnp.abs(n)
    max_exact = half // 2
    is_small = n < max_exact
    nf = n.astype(jnp.float32)
    val_if_large = max_exact + (
        jnp.log(nf / max_exact)
        / math.log(_MAX_DISTANCE / max_exact)
        * (half - max_exact)
    ).astype(jnp.int32)
    val_if_large = jnp.minimum(val_if_large, half - 1)
    bucket = ret + jnp.where(is_small, n, val_if_large)

    onehot = jnp.equal(
        lax.broadcasted_iota(jnp.int32, (_NUM_BUCKETS, _TEXT), 0), bucket
    ).astype(jnp.float32)
    t_ext = lax.dot_general(
        w_ref[...], onehot, (((0,), (0,)), ((), ())),
        preferred_element_type=jnp.float32,
        precision=lax.Precision.HIGHEST,
    )
    for k in range(_NSHIFT):
        sh = _NSHIFT - 1 - k
        out_ref[:, k, :] = t_ext[:, sh:sh + _TWIDTH]


def _make_table(weight, delta):
    return pl.pallas_call(
        _table_body,
        out_shape=jax.ShapeDtypeStruct((_N_HEADS, _NSHIFT, _TWIDTH), jnp.float32),
        in_specs=[
            pl.BlockSpec(memory_space=pltpu.SMEM),
            pl.BlockSpec(memory_space=pltpu.VMEM),
        ],
        out_specs=pl.BlockSpec(memory_space=pltpu.VMEM),
    )(delta, weight)


_GROUPS_PER_TILE = 1024 // _NSHIFT


def _writer_body(u_hbm, out_hbm, u_v, sem):
    wid = lax.axis_index("s") * 2 + lax.axis_index("c")
    head = wid // 2
    g0 = (wid % 2) * _GROUPS_PER_TILE
    pltpu.sync_copy(u_hbm.at[head], u_v)

    def fire(i, carry):
        g = g0 + i
        base = pl.multiple_of((_QLEN - _NSHIFT) - _NSHIFT * g, _NSHIFT)
        pltpu.make_async_copy(
            u_v.at[:, pl.ds(base, _KLEN)],
            out_hbm.at[0, head, pl.ds(_NSHIFT * g, _NSHIFT)],
            sem,
        ).start()
        return carry

    lax.fori_loop(0, _GROUPS_PER_TILE, fire, 0)

    def drain(i, carry):
        pltpu.make_async_copy(
            u_v.at[:, pl.ds(0, _KLEN)],
            out_hbm.at[0, head, pl.ds(0, _NSHIFT)],
            sem,
        ).wait()
        return carry

    lax.fori_loop(0, _GROUPS_PER_TILE, drain, 0)


@functools.cache
def _writer():
    return pl.kernel(
        _writer_body,
        out_type=jax.ShapeDtypeStruct((1, _N_HEADS, _QLEN, _KLEN), jnp.float32),
        mesh=plsc.VectorSubcoreMesh(core_axis_name="c", subcore_axis_name="s"),
        scratch_types=[
            pltpu.VMEM((_NSHIFT, _TWIDTH), jnp.float32),
            pltpu.SemaphoreType.DMA,
        ],
        compiler_params=pltpu.CompilerParams(use_tc_tiling_on_sc=False),
    )


def kernel(weight, qlen, klen):
    delta = (jnp.asarray(klen, jnp.int32) - jnp.asarray(qlen, jnp.int32))
    t16 = _make_table(weight, delta.reshape(1, 1))
    return _writer()(t16)


# speedup vs baseline: 2.8066x; 2.8066x over previous
import functools
import math

import jax
import jax.numpy as jnp
from jax import lax
from jax.experimental import pallas as pl
from jax.experimental.pallas import tpu as pltpu
from jax.experimental.pallas import tpu_sc as plsc

_NUM_BUCKETS = 32
_MAX_DISTANCE = 128
_N_HEADS = 16
_QLEN = 2048
_KLEN = 2048
_TEXT = 4224
_SROWS = 128
_SWIDTH = 4096


def _table_body(delta_ref, w_ref, out_ref, t_ref, x_ref):
    h = pl.program_id(0)

    @pl.when(h == 0)
    def _compute_t_ext():
        m = lax.broadcasted_iota(jnp.int32, (1, _TEXT), 1)
        rel = m - (_QLEN - 1) + delta_ref[0, 0]
        n = -rel
        half = _NUM_BUCKETS // 2
        ret = jnp.where(n < 0, half, 0).astype(jnp.int32)
        n = jnp.abs(n)
        max_exact = half // 2
        is_small = n < max_exact
        nf = n.astype(jnp.float32)
        val_if_large = max_exact + (
            jnp.log(nf / max_exact)
            / math.log(_MAX_DISTANCE / max_exact)
            * (half - max_exact)
        ).astype(jnp.int32)
        val_if_large = jnp.minimum(val_if_large, half - 1)
        bucket = ret + jnp.where(is_small, n, val_if_large)
        onehot = jnp.equal(
            lax.broadcasted_iota(jnp.int32, (_NUM_BUCKETS, _TEXT), 0), bucket
        ).astype(jnp.float32)
        t_ref[...] = lax.dot_general(
            w_ref[...], onehot, (((0,), (0,)), ((), ())),
            preferred_element_type=jnp.float32,
            precision=lax.Precision.HIGHEST,
        )

    for u in range(8):
        x_ref[u:u + 1, 0:4216] = t_ref[pl.ds(h, 1), 7 - u:7 - u + 4216]
    for p in range(16):
        out_ref[0, 8 * p:8 * p + 8, :] = x_ref[:, 120 - 8 * p:120 - 8 * p + _SWIDTH]


def _make_stable(weight, delta):
    return pl.pallas_call(
        _table_body,
        grid=(_N_HEADS,),
        out_shape=jax.ShapeDtypeStruct((_N_HEADS, _SROWS, _SWIDTH), jnp.float32),
        in_specs=[
            pl.BlockSpec(memory_space=pltpu.SMEM),
            pl.BlockSpec(memory_space=pltpu.VMEM),
        ],
        out_specs=pl.BlockSpec((1, _SROWS, _SWIDTH), lambda h: (h, 0, 0)),
        scratch_shapes=[
            pltpu.VMEM((_N_HEADS, _TEXT), jnp.float32),
            pltpu.VMEM((8, _TEXT), jnp.float32),
        ],
    )(delta, weight)


_HEADS_PER_CORE = _N_HEADS // 2


def _writer_body(s_hbm, out_hbm, tv, sem_fire, sem_stage):
    c = lax.axis_index("c")
    s = lax.axis_index("s")
    row8 = pl.multiple_of(8 * s, 8)

    def start_stage(k, buf):
        pltpu.make_async_copy(
            s_hbm.at[c * _HEADS_PER_CORE + k, pl.ds(row8, 8), :],
            tv.at[buf], sem_stage,
        ).start()

    def wait_stage():
        pltpu.make_async_copy(
            s_hbm.at[0, pl.ds(0, 8), :], tv.at[0], sem_stage
        ).wait()

    start_stage(0, 0)
    wait_stage()
    for k in range(_HEADS_PER_CORE):
        h = c * _HEADS_PER_CORE + k
        if k + 1 < _HEADS_PER_CORE:
            start_stage(k + 1, (k + 1) % 2)
        buf = k % 2
        for t in range(16):
            q = 15 - t
            pltpu.make_async_copy(
                tv.at[buf, :, pl.ds(128 * q, _KLEN)],
                out_hbm.at[0, h, pl.ds(128 * t + row8, 8), :],
                sem_fire,
            ).start()
        for t in range(16):
            pltpu.make_async_copy(
                tv.at[0, :, pl.ds(0, _KLEN)],
                out_hbm.at[0, 0, pl.ds(0, 8), :],
                sem_fire,
            ).wait()
        if k + 1 < _HEADS_PER_CORE:
            wait_stage()


@functools.cache
def _writer():
    return pl.kernel(
        _writer_body,
        out_type=jax.ShapeDtypeStruct((1, _N_HEADS, _QLEN, _KLEN), jnp.float32),
        mesh=plsc.VectorSubcoreMesh(core_axis_name="c", subcore_axis_name="s"),
        scratch_types=[
            pltpu.VMEM((2, 8, _SWIDTH), jnp.float32),
            pltpu.SemaphoreType.DMA,
            pltpu.SemaphoreType.DMA,
        ],
        compiler_params=pltpu.CompilerParams(use_tc_tiling_on_sc=True),
    )


def kernel(weight, qlen, klen):
    delta = (jnp.asarray(klen, jnp.int32) - jnp.asarray(qlen, jnp.int32))
    stable = _make_stable(weight, delta.reshape(1, 1))
    return _writer()(stable)


# speedup vs baseline: 2.8123x; 1.0020x over previous
import functools
import math

import jax
import jax.numpy as jnp
from jax import lax
from jax.experimental import pallas as pl
from jax.experimental.pallas import tpu as pltpu
from jax.experimental.pallas import tpu_sc as plsc

_NUM_BUCKETS = 32
_MAX_DISTANCE = 128
_N_HEADS = 16
_QLEN = 2048
_KLEN = 2048
_TEXT = 4224
_SROWS = 128
_SWIDTH = 4096


def _table_body(delta_ref, w_ref, out_ref, t_ref, x_ref):
    h = pl.program_id(0)

    @pl.when(h == 0)
    def _compute_t_ext():
        m = lax.broadcasted_iota(jnp.int32, (1, _TEXT), 1)
        rel = m - (_QLEN - 1) + delta_ref[0, 0]
        n = -rel
        half = _NUM_BUCKETS // 2
        ret = jnp.where(n < 0, half, 0).astype(jnp.int32)
        n = jnp.abs(n)
        max_exact = half // 2
        is_small = n < max_exact
        nf = n.astype(jnp.float32)
        val_if_large = max_exact + (
            jnp.log(nf / max_exact)
            / math.log(_MAX_DISTANCE / max_exact)
            * (half - max_exact)
        ).astype(jnp.int32)
        val_if_large = jnp.minimum(val_if_large, half - 1)
        bucket = ret + jnp.where(is_small, n, val_if_large)
        onehot = jnp.equal(
            lax.broadcasted_iota(jnp.int32, (_NUM_BUCKETS, _TEXT), 0), bucket
        ).astype(jnp.float32)
        t_ref[...] = lax.dot_general(
            w_ref[...], onehot, (((0,), (0,)), ((), ())),
            preferred_element_type=jnp.float32,
            precision=lax.Precision.HIGHEST,
        )

    for u in range(8):
        x_ref[u:u + 1, 0:4216] = t_ref[pl.ds(h, 1), 7 - u:7 - u + 4216]
    for p in range(16):
        out_ref[0, 8 * p:8 * p + 8, :] = x_ref[:, 120 - 8 * p:120 - 8 * p + _SWIDTH]


def _make_stable(weight, delta):
    return pl.pallas_call(
        _table_body,
        grid=(_N_HEADS,),
        out_shape=jax.ShapeDtypeStruct((_N_HEADS, _SROWS, _SWIDTH), jnp.float32),
        in_specs=[
            pl.BlockSpec(memory_space=pltpu.SMEM),
            pl.BlockSpec(memory_space=pltpu.VMEM),
        ],
        out_specs=pl.BlockSpec((1, _SROWS, _SWIDTH), lambda h: (h, 0, 0)),
        scratch_shapes=[
            pltpu.VMEM((_N_HEADS, _TEXT), jnp.float32),
            pltpu.VMEM((8, _TEXT), jnp.float32),
        ],
    )(delta, weight)


_HEADS_PER_CORE = _N_HEADS // 2


def _writer_body(s_hbm, out_hbm, tv, sem_fire, sem_stage):
    c = lax.axis_index("c")
    s = lax.axis_index("s")
    row8 = pl.multiple_of(8 * s, 8)

    def start_stage(k, buf):
        pltpu.make_async_copy(
            s_hbm.at[c * _HEADS_PER_CORE + k, pl.ds(row8, 8), :],
            tv.at[buf], sem_stage,
        ).start()

    def wait_stage():
        pltpu.make_async_copy(
            s_hbm.at[0, pl.ds(0, 8), :], tv.at[0], sem_stage
        ).wait()

    def drain_fires():
        for _ in range(16):
            pltpu.make_async_copy(
                tv.at[0, :, pl.ds(0, _KLEN)],
                out_hbm.at[0, 0, pl.ds(0, 8), :],
                sem_fire,
            ).wait()

    start_stage(0, 0)
    wait_stage()
    for k in range(_HEADS_PER_CORE):
        h = c * _HEADS_PER_CORE + k
        if k > 0:
            drain_fires()
        if k + 1 < _HEADS_PER_CORE:
            start_stage(k + 1, (k + 1) % 2)
        buf = k % 2
        for t in range(16):
            q = 15 - t
            pltpu.make_async_copy(
                tv.at[buf, :, pl.ds(128 * q, _KLEN)],
                out_hbm.at[0, h, pl.ds(128 * t + row8, 8), :],
                sem_fire,
            ).start()
        if k + 1 < _HEADS_PER_CORE:
            wait_stage()
    drain_fires()


@functools.cache
def _writer():
    return pl.kernel(
        _writer_body,
        out_type=jax.ShapeDtypeStruct((1, _N_HEADS, _QLEN, _KLEN), jnp.float32),
        mesh=plsc.VectorSubcoreMesh(core_axis_name="c", subcore_axis_name="s"),
        scratch_types=[
            pltpu.VMEM((2, 8, _SWIDTH), jnp.float32),
            pltpu.SemaphoreType.DMA,
            pltpu.SemaphoreType.DMA,
        ],
        compiler_params=pltpu.CompilerParams(use_tc_tiling_on_sc=True),
    )


def kernel(weight, qlen, klen):
    delta = (jnp.asarray(klen, jnp.int32) - jnp.asarray(qlen, jnp.int32))
    stable = _make_stable(weight, delta.reshape(1, 1))
    return _writer()(stable)


# speedup vs baseline: 2.8498x; 1.0134x over previous
import functools
import math

import jax
import jax.numpy as jnp
from jax import lax
from jax.experimental import pallas as pl
from jax.experimental.pallas import tpu as pltpu
from jax.experimental.pallas import tpu_sc as plsc

_NUM_BUCKETS = 32
_MAX_DISTANCE = 128
_N_HEADS = 16
_QLEN = 2048
_KLEN = 2048
_TEXT = 4224
_SROWS = 128
_SWIDTH = 4096


def _table_body(delta_ref, w_ref, out_ref, t_ref, x_ref):
    h = pl.program_id(0)

    @pl.when(h == 0)
    def _compute_t_ext():
        m = lax.broadcasted_iota(jnp.int32, (1, _TEXT), 1)
        rel = m - (_QLEN - 1) + delta_ref[0, 0]
        n = -rel
        half = _NUM_BUCKETS // 2
        ret = jnp.where(n < 0, half, 0).astype(jnp.int32)
        n = jnp.abs(n)
        max_exact = half // 2
        is_small = n < max_exact
        nf = n.astype(jnp.float32)
        val_if_large = max_exact + (
            jnp.log(nf / max_exact)
            / math.log(_MAX_DISTANCE / max_exact)
            * (half - max_exact)
        ).astype(jnp.int32)
        val_if_large = jnp.minimum(val_if_large, half - 1)
        bucket = ret + jnp.where(is_small, n, val_if_large)
        onehot = jnp.equal(
            lax.broadcasted_iota(jnp.int32, (_NUM_BUCKETS, _TEXT), 0), bucket
        ).astype(jnp.float32)
        t_ref[...] = lax.dot_general(
            w_ref[...], onehot, (((0,), (0,)), ((), ())),
            preferred_element_type=jnp.float32,
            precision=lax.Precision.HIGHEST,
        )

    for u in range(8):
        x_ref[u:u + 1, 0:4216] = t_ref[pl.ds(h, 1), 7 - u:7 - u + 4216]
    for p in range(16):
        out_ref[0, 8 * p:8 * p + 8, :] = x_ref[:, 120 - 8 * p:120 - 8 * p + _SWIDTH]


def _make_stable(weight, delta):
    return pl.pallas_call(
        _table_body,
        grid=(_N_HEADS,),
        out_shape=jax.ShapeDtypeStruct((_N_HEADS, _SROWS, _SWIDTH), jnp.float32),
        in_specs=[
            pl.BlockSpec(memory_space=pltpu.SMEM),
            pl.BlockSpec(memory_space=pltpu.VMEM),
        ],
        out_specs=pl.BlockSpec((1, _SROWS, _SWIDTH), lambda h: (h, 0, 0)),
        scratch_shapes=[
            pltpu.VMEM((_N_HEADS, _TEXT), jnp.float32),
            pltpu.VMEM((8, _TEXT), jnp.float32),
        ],
    )(delta, weight)


_HEADS_PER_CORE = _N_HEADS // 2


def _writer_body(s_hbm, out_hbm, tv, sem_fire, sem_stage):
    c = lax.axis_index("c")
    s = lax.axis_index("s")
    row8 = pl.multiple_of(8 * s, 8)

    def start_stage(k, buf):
        pltpu.make_async_copy(
            s_hbm.at[c * _HEADS_PER_CORE + k, pl.ds(row8, 8), :],
            tv.at[buf], sem_stage,
        ).start()

    def wait_stage():
        pltpu.make_async_copy(
            s_hbm.at[0, pl.ds(0, 8), :], tv.at[0], sem_stage
        ).wait()

    def drain_fires():
        def body(t, carry):
            pltpu.make_async_copy(
                tv.at[0, :, pl.ds(0, _KLEN)],
                out_hbm.at[0, 0, pl.ds(0, 8), :],
                sem_fire,
            ).wait()
            return carry

        lax.fori_loop(0, 16, body, 0)

    def fire_head(k):
        h = c * _HEADS_PER_CORE + k
        buf = jnp.remainder(k, 2)

        def body(t, carry):
            q = pl.multiple_of(128 * (15 - t), 128)
            pltpu.make_async_copy(
                tv.at[buf, :, pl.ds(q, _KLEN)],
                out_hbm.at[0, h, pl.ds(pl.multiple_of(128 * t + row8, 8), 8), :],
                sem_fire,
            ).start()
            return carry

        lax.fori_loop(0, 16, body, 0)

    start_stage(0, 0)
    wait_stage()

    def head_body(k, carry):
        @pl.when(k > 0)
        def _():
            drain_fires()

        @pl.when(k + 1 < _HEADS_PER_CORE)
        def _():
            start_stage(k + 1, jnp.remainder(k + 1, 2))

        fire_head(k)

        @pl.when(k + 1 < _HEADS_PER_CORE)
        def _():
            wait_stage()

        return carry

    lax.fori_loop(0, _HEADS_PER_CORE, head_body, 0)
    drain_fires()


@functools.cache
def _writer():
    return pl.kernel(
        _writer_body,
        out_type=jax.ShapeDtypeStruct((1, _N_HEADS, _QLEN, _KLEN), jnp.float32),
        mesh=plsc.VectorSubcoreMesh(core_axis_name="c", subcore_axis_name="s"),
        scratch_types=[
            pltpu.VMEM((2, 8, _SWIDTH), jnp.float32),
            pltpu.SemaphoreType.DMA,
            pltpu.SemaphoreType.DMA,
        ],
        compiler_params=pltpu.CompilerParams(use_tc_tiling_on_sc=True),
    )


def kernel(weight, qlen, klen):
    delta = (jnp.asarray(klen, jnp.int32) - jnp.asarray(qlen, jnp.int32))
    stable = _make_stable(weight, delta.reshape(1, 1))
    return _writer()(stable)
